# baseline (speedup 1.0000x reference)
import functools

import jax
import jax.numpy as jnp
from jax import lax
from jax.experimental import pallas as pl
from jax.experimental.pallas import tpu as pltpu
from jax.experimental.pallas import tpu_sc as plsc

_D = 128
_WINDOW = 128
_WMUL = 2
_NTBL = 1024


def _prep_body(d_ids_ref, s_ids_ref, dt_ref, st_ref, t3_ref, cidx_ref):
    cidx_ref[...] = d_ids_ref[...] * 50 + s_ids_ref[...]
    dt = dt_ref[...]
    st = st_ref[...]
    t3_ref[...] = dt[:, None, :] + st[None, :, :]


def _sc_gather(table, cidx, n):
    mesh = plsc.VectorSubcoreMesh(core_axis_name="c", subcore_axis_name="s")

    @functools.partial(
        pl.kernel,
        out_type=jax.ShapeDtypeStruct((n, _D), jnp.float32),
        mesh=mesh,
        scratch_types=[pltpu.VMEM_SHARED((1000, _D), jnp.float32)],
    )
    def k(tbl_hbm, idx_hbm, out_hbm, tbl_sh):
        @pl.when(lax.axis_index("s") == 0)
        def _():
            pltpu.sync_copy(tbl_hbm, tbl_sh)

        plsc.subcore_barrier()

        def body(i_vmem, o_vmem):
            for j in range(_WMUL):
                pltpu.sync_copy(
                    tbl_sh.at[i_vmem.at[j]],
                    o_vmem.at[pl.ds(j * _WINDOW, _WINDOW)],
                )

        pltpu.emit_pipeline(
            body,
            grid=(n // (_WMUL * _WINDOW),),
            in_specs=[pl.BlockSpec((_WMUL, _WINDOW), lambda i: (i, 0))],
            out_specs=[pl.BlockSpec((_WMUL * _WINDOW, _D), lambda i: (i, 0))],
            core_axis_name=("c", "s"),
            dimension_semantics=(pltpu.PARALLEL,),
        )(idx_hbm, out_hbm)

    return k(table, cidx)


def kernel(depth_ids, subtree_ids, depth_table, subtree_table):
    b, sq = depth_ids.shape
    nd, d_model = depth_table.shape
    ns = subtree_table.shape[0]
    n = b * sq

    d_ids2 = depth_ids.reshape(n // 128, 128).astype(jnp.int32)
    s_ids2 = subtree_ids.reshape(n // 128, 128).astype(jnp.int32)

    t3, cidx2 = pl.pallas_call(
        _prep_body,
        out_shape=[
            jax.ShapeDtypeStruct((nd, ns, d_model), jnp.float32),
            jax.ShapeDtypeStruct((n // 128, 128), jnp.int32),
        ],
    )(d_ids2, s_ids2, depth_table, subtree_table)

    table = t3.reshape(nd * ns, d_model)

    out = _sc_gather(table, cidx2, n)
    return out.reshape(b, sq, d_model)
